# baseline (speedup 1.0000x reference)
import jax
import jax.numpy as jnp
from jax.experimental import pallas as pl
from jax.experimental.pallas import tpu as pltpu

_B, _N, _F, _C = 8, 2048, 64, 64
_M = _B * _N
_P = 4
_R = _M // _P
_L = _P * _C
_RT = _R // _B
_INV_SQRT2 = 0.7071067811865476
_POST = 2.0 ** 0.5 / 2.0


def _stats_to_scale_offset(s1, s2, g_ref, b_ref):
    s1 = sum(s1[:, i * _C:(i + 1) * _C] for i in range(_P)) * (1.0 / _M)
    s2 = sum(s2[:, i * _C:(i + 1) * _C] for i in range(_P)) * (1.0 / _M)
    var = s2 - s1 * s1
    scale = jax.lax.rsqrt(var + 1e-5) * g_ref[...] * _INV_SQRT2
    offset = b_ref[...] * _INV_SQRT2 - s1 * scale
    scale = jnp.concatenate([scale] * _P, axis=1)
    offset = jnp.concatenate([offset] * _P, axis=1)
    return scale, offset


def _gelu_from_t(t):
    return t * (1.0 + jax.lax.erf(t)) * _POST


def _mlp_bn_kernel(x_hbm, w1_ref, g1_ref, b1_ref, w2_ref, g2_ref, b2_ref,
                   w3_ref, g3_ref, b3_ref, out_hbm, xv, out_v, sems, osems):
    copies = [
        pltpu.make_async_copy(
            x_hbm.at[t * _N:(t + 1) * _N, :],
            xv.at[t * _N:(t + 1) * _N, :],
            sems.at[t])
        for t in range(_B)
    ]
    for c in copies:
        c.start()

    h1_tiles = []
    s1 = jnp.zeros((1, _L), jnp.float32)
    s2 = jnp.zeros((1, _L), jnp.float32)
    for t in range(_B):
        copies[t].wait()
        base = t * _N
        at = jnp.concatenate(
            [xv[base + j * _RT:base + (j + 1) * _RT, :] for j in range(_P)],
            axis=1)
        hm = jax.lax.dot_general(at.astype(jnp.bfloat16), w1_ref[...],
                                 (((1,), (0,)), ((), ())),
                                 preferred_element_type=jnp.float32)
        h1_tiles.append(hm)
        s1 = s1 + jnp.sum(hm, axis=0, keepdims=True)
        s2 = s2 + jnp.sum(hm * hm, axis=0, keepdims=True)

    scale, offset = _stats_to_scale_offset(s1, s2, g1_ref, b1_ref)
    a = _gelu_from_t(jnp.concatenate(h1_tiles, axis=0) * scale + offset)

    def layer(h, w_ref, g_ref, b_ref):
        hm = jax.lax.dot_general(h.astype(jnp.bfloat16), w_ref[...],
                                 (((1,), (0,)), ((), ())),
                                 preferred_element_type=jnp.float32)
        ls1 = jnp.sum(hm, axis=0, keepdims=True)
        ls2 = jnp.sum(hm * hm, axis=0, keepdims=True)
        sc, of = _stats_to_scale_offset(ls1, ls2, g_ref, b_ref)
        return _gelu_from_t(hm * sc + of)

    a = layer(a, w2_ref, g2_ref, b2_ref)
    a = layer(a, w3_ref, g3_ref, b3_ref)

    for t in range(_B):
        for j in range(_P):
            out_v[t, :, j * _RT:(j + 1) * _RT] = jnp.transpose(
                a[t * _RT:(t + 1) * _RT, j * _C:(j + 1) * _C], (1, 0))
        ocopy = pltpu.make_async_copy(out_v.at[t], out_hbm.at[t], osems.at[t])
        ocopy.start()
    for t in range(_B):
        pltpu.make_async_copy(out_v.at[t], out_hbm.at[t], osems.at[t]).wait()


def _blockdiag(W):
    return jnp.kron(jnp.eye(_P, dtype=W.dtype), W.T).astype(jnp.bfloat16)


def kernel(x, W1, g1, b1, W2, g2, b2, W3, g3, b3):
    xp = x.reshape(_M, _F)
    vspec = pl.BlockSpec(memory_space=pltpu.MemorySpace.VMEM)
    aspec = pl.BlockSpec(memory_space=pl.MemorySpace.ANY)
    return pl.pallas_call(
        _mlp_bn_kernel,
        in_specs=[aspec] + [vspec] * 9,
        out_specs=aspec,
        out_shape=jax.ShapeDtypeStruct((_B, _C, _N), jnp.float32),
        scratch_shapes=[
            pltpu.VMEM((_M, _F), jnp.float32),
            pltpu.VMEM((_B, _C, _N), jnp.float32),
            pltpu.SemaphoreType.DMA((_B,)),
            pltpu.SemaphoreType.DMA((_B,)),
        ],
    )(xp, _blockdiag(W1), g1.reshape(1, _C), b1.reshape(1, _C),
      _blockdiag(W2), g2.reshape(1, _C), b2.reshape(1, _C),
      _blockdiag(W3), g3.reshape(1, _C), b3.reshape(1, _C))
